# Initial kernel scaffold; baseline (speedup 1.0000x reference)
import jax
import jax.numpy as jnp
from jax.experimental import pallas as pl


def kernel(x):
    raise NotImplementedError("write your pallas kernel here")


# speedup vs baseline: 3.9214x; 3.9214x over previous
import functools

import jax
import jax.numpy as jnp
import numpy as np
from jax.experimental import pallas as pl

_M0 = 0.01
_LIMS = [[-1.0, 1.0], [-1.0, 1.0]]
_SIZE = [128, 128]


def _dtm_body(gx_ref, gy_ref, x0_ref, x1_ref, out_ref, *, k, bound, cum_w):
    gx = gx_ref[...]
    gy = gy_ref[...]
    x0 = x0_ref[0:1, :]
    x1 = x1_ref[0:1, :]

    d2 = (gx - x0) ** 2 + (gy - x1) ** 2

    r = d2.shape[0]
    inf = jnp.float32(jnp.inf)
    s = jnp.zeros((r, 1), jnp.float32)
    kth = jnp.zeros((r, 1), jnp.float32)
    rem = jnp.full((r, 1), jnp.float32(k))

    for _ in range(k):
        m = jnp.min(d2, axis=1, keepdims=True)
        eq = d2 == m
        c = jnp.sum(eq.astype(jnp.float32), axis=1, keepdims=True)
        take = jnp.minimum(c, rem)
        s = s + take * m
        kth = jnp.where((rem > 0) & (c >= rem), m, kth)
        rem = rem - take
        d2 = jnp.where(eq, inf, d2)

    val = s + kth * jnp.float32(bound - cum_w)
    out_ref[...] = jnp.sqrt(val * jnp.float32(1.0 / bound))


@jax.jit
def kernel(x):
    n = x.shape[-2]
    bound = _M0 * n
    k = int(np.ceil(bound))
    cum_w = float(np.ceil(bound))

    x_seq = jnp.linspace(_LIMS[0][0], _LIMS[0][1], _SIZE[0])
    y_seq = jnp.linspace(_LIMS[1][1], _LIMS[1][0], _SIZE[1])
    x_coord, y_coord = jnp.meshgrid(x_seq, y_seq, indexing='xy')
    gx = x_coord.reshape(-1, 1).astype(jnp.float32)
    gy = y_coord.reshape(-1, 1).astype(jnp.float32)

    hw = _SIZE[0] * _SIZE[1]
    rows = 1024

    x0 = jnp.broadcast_to(x[:, 0].reshape(1, n), (8, n))
    x1 = jnp.broadcast_to(x[:, 1].reshape(1, n), (8, n))

    out = pl.pallas_call(
        functools.partial(_dtm_body, k=k, bound=bound, cum_w=cum_w),
        grid=(hw // rows,),
        in_specs=[
            pl.BlockSpec((rows, 1), lambda i: (i, 0)),
            pl.BlockSpec((rows, 1), lambda i: (i, 0)),
            pl.BlockSpec((8, n), lambda i: (0, 0)),
            pl.BlockSpec((8, n), lambda i: (0, 0)),
        ],
        out_specs=pl.BlockSpec((rows, 1), lambda i: (i, 0)),
        out_shape=jax.ShapeDtypeStruct((hw, 1), jnp.float32),
    )(gx, gy, x0, x1)

    return out.reshape(_SIZE[0], _SIZE[1])


# speedup vs baseline: 7.7263x; 1.9703x over previous
import functools

import jax
import jax.numpy as jnp
import numpy as np
from jax.experimental import pallas as pl

_M0 = 0.01
_LIMS = [[-1.0, 1.0], [-1.0, 1.0]]
_SIZE = [128, 128]

_LEVELS = 8

_SORT8 = [
    (0, 1), (2, 3), (0, 2), (1, 3), (1, 2),
    (4, 5), (6, 7), (4, 6), (5, 7), (5, 6),
    (0, 4), (1, 5), (2, 6), (3, 7),
    (2, 4), (3, 5),
    (1, 2), (3, 4), (5, 6),
]


def _dtm_body(gx_ref, gy_ref, x0_ref, x1_ref, out_ref, *, k, bound, cum_w):
    gx = gx_ref[...]
    gy = gy_ref[...]
    x0 = x0_ref[0:1, :]
    x1 = x1_ref[0:1, :]

    n = x0.shape[1]
    w = n // _LEVELS

    vals = []
    for t in range(_LEVELS):
        sl = slice(t * w, (t + 1) * w)
        vals.append((gx - x0[:, sl]) ** 2 + (gy - x1[:, sl]) ** 2)

    for a, b in _SORT8:
        lo = jnp.minimum(vals[a], vals[b])
        hi = jnp.maximum(vals[a], vals[b])
        vals[a], vals[b] = lo, hi

    r = gx.shape[0]
    inf = jnp.float32(jnp.inf)
    s = jnp.zeros((r, 1), jnp.float32)
    kth = jnp.zeros((r, 1), jnp.float32)
    rem = jnp.full((r, 1), jnp.float32(k))

    for _ in range(k):
        front = vals[0]
        m = jnp.min(front, axis=1, keepdims=True)
        hit = front == m
        c = jnp.sum(hit.astype(jnp.float32), axis=1, keepdims=True)
        take = jnp.minimum(c, rem)
        s = s + take * jnp.where(take > 0, m, jnp.float32(0.0))
        kth = jnp.where((rem > 0) & (c >= rem), m, kth)
        rem = rem - take
        for i in range(_LEVELS - 1):
            vals[i] = jnp.where(hit, vals[i + 1], vals[i])
        vals[_LEVELS - 1] = jnp.where(hit, inf, vals[_LEVELS - 1])

    val = s + kth * jnp.float32(bound - cum_w)
    out_ref[...] = jnp.sqrt(val * jnp.float32(1.0 / bound))


@jax.jit
def kernel(x):
    n = x.shape[-2]
    bound = _M0 * n
    k = int(np.ceil(bound))
    cum_w = float(np.ceil(bound))

    x_seq = jnp.linspace(_LIMS[0][0], _LIMS[0][1], _SIZE[0])
    y_seq = jnp.linspace(_LIMS[1][1], _LIMS[1][0], _SIZE[1])
    x_coord, y_coord = jnp.meshgrid(x_seq, y_seq, indexing='xy')
    gx = x_coord.reshape(-1, 1).astype(jnp.float32)
    gy = y_coord.reshape(-1, 1).astype(jnp.float32)

    hw = _SIZE[0] * _SIZE[1]
    rows = 1024

    x0 = jnp.broadcast_to(x[:, 0].reshape(1, n), (8, n))
    x1 = jnp.broadcast_to(x[:, 1].reshape(1, n), (8, n))

    out = pl.pallas_call(
        functools.partial(_dtm_body, k=k, bound=bound, cum_w=cum_w),
        grid=(hw // rows,),
        in_specs=[
            pl.BlockSpec((rows, 1), lambda i: (i, 0)),
            pl.BlockSpec((rows, 1), lambda i: (i, 0)),
            pl.BlockSpec((8, n), lambda i: (0, 0)),
            pl.BlockSpec((8, n), lambda i: (0, 0)),
        ],
        out_specs=pl.BlockSpec((rows, 1), lambda i: (i, 0)),
        out_shape=jax.ShapeDtypeStruct((hw, 1), jnp.float32),
    )(gx, gy, x0, x1)

    return out.reshape(_SIZE[0], _SIZE[1])
